# Initial kernel scaffold; baseline (speedup 1.0000x reference)
import jax
import jax.numpy as jnp
from jax.experimental import pallas as pl


def kernel(input_ids, word_table, pos_table, tt_table, ln_gamma, ln_beta, bias_subspace):
    raise NotImplementedError("write your pallas kernel here")


# speedup vs baseline: 1.0070x; 1.0070x over previous
import functools

import jax
import jax.numpy as jnp
from jax import lax
from jax.experimental import pallas as pl
from jax.experimental.pallas import tpu as pltpu
from jax.experimental.pallas import tpu_sc as plsc

_EPS = 1e-12


def _sc_gather(table, ids3, nw, nch, ch, h):
    mesh = plsc.VectorSubcoreMesh(core_axis_name="c", subcore_axis_name="s")
    info = plsc.get_sparse_core_info()
    nc = info.num_cores

    @functools.partial(
        pl.kernel,
        mesh=mesh,
        out_type=jax.ShapeDtypeStruct((nw * nch * ch, h), jnp.float32),
        scratch_types=[
            pltpu.VMEM((nch, ch), jnp.int32),
            pltpu.VMEM((ch, h), jnp.float32),
            pltpu.VMEM((ch, h), jnp.float32),
            pltpu.SemaphoreType.DMA,
            pltpu.SemaphoreType.DMA,
        ],
    )
    def gather_kernel(table_hbm, ids_hbm, out_hbm, idx_v, rows0, rows1, sem0, sem1):
        wid = lax.axis_index("s") * nc + lax.axis_index("c")
        pltpu.sync_copy(ids_hbm.at[wid], idx_v)
        bufs = (rows0, rows1)
        sems = (sem0, sem1)
        for c in range(nch):
            buf = bufs[c % 2]
            sem = sems[c % 2]
            pltpu.async_copy(table_hbm.at[idx_v.at[c]], buf, sem).wait()
            pltpu.sync_copy(buf, out_hbm.at[pl.ds(wid * nch * ch + c * ch, ch)])

    return gather_kernel(table, ids3)


def _col_sumsq(table, v, h, vb):
    g = v // vb

    def body(x_ref, o_ref):
        i = pl.program_id(0)

        @pl.when(i == 0)
        def _():
            o_ref[...] = jnp.zeros_like(o_ref)

        x = x_ref[...]
        o_ref[...] += jnp.sum(x * x, axis=0, keepdims=True)

    return pl.pallas_call(
        body,
        grid=(g,),
        in_specs=[pl.BlockSpec((vb, h), lambda i: (i, 0))],
        out_specs=pl.BlockSpec((1, h), lambda i: (0, 0)),
        out_shape=jax.ShapeDtypeStruct((1, h), jnp.float32),
    )(table)


def _pointwise(rows, norm2, pos_table, tt_row, gamma, beta, bias, n, s, h, tb):
    g = n // tb
    pb = s // tb

    def body(r_ref, n2_ref, p_ref, tt_ref, g_ref, b_ref, bias_ref, o_ref):
        x = r_ref[...]
        nn = jnp.sqrt(n2_ref[...])
        x = x - bias_ref[...] - nn
        inv = lax.rsqrt(jnp.sum(x * x, axis=-1, keepdims=True))
        x = x * inv + p_ref[...] + tt_ref[...]
        m = jnp.mean(x, axis=-1, keepdims=True)
        xc = x - m
        var = jnp.mean(xc * xc, axis=-1, keepdims=True)
        o_ref[...] = xc * lax.rsqrt(var + _EPS) * g_ref[...] + b_ref[...]

    return pl.pallas_call(
        body,
        grid=(g,),
        in_specs=[
            pl.BlockSpec((tb, h), lambda i: (i, 0)),
            pl.BlockSpec((1, h), lambda i: (0, 0)),
            pl.BlockSpec((tb, h), lambda i: (i % pb, 0)),
            pl.BlockSpec((1, h), lambda i: (0, 0)),
            pl.BlockSpec((1, h), lambda i: (0, 0)),
            pl.BlockSpec((1, h), lambda i: (0, 0)),
            pl.BlockSpec((1, h), lambda i: (0, 0)),
        ],
        out_specs=pl.BlockSpec((tb, h), lambda i: (i, 0)),
        out_shape=jax.ShapeDtypeStruct((n, h), jnp.float32),
    )(rows, norm2, pos_table, tt_row, gamma, beta, bias)


def kernel(input_ids, word_table, pos_table, tt_table, ln_gamma, ln_beta, bias_subspace):
    b, s = input_ids.shape
    v, h = word_table.shape
    n = b * s

    nw = 32
    ch = 64
    nch = n // (nw * ch)
    ids3 = input_ids.reshape(nw, nch, ch).astype(jnp.int32)

    gathered = _sc_gather(word_table, ids3, nw, nch, ch, h)
    norm2 = _col_sumsq(word_table, v, h, vb=2000)
    out = _pointwise(
        gathered,
        norm2,
        pos_table,
        tt_table[0:1],
        ln_gamma.reshape(1, h),
        ln_beta.reshape(1, h),
        bias_subspace.reshape(1, h),
        n,
        s,
        h,
        tb=256,
    )
    return out.reshape(b, s, h)
